# Initial kernel scaffold; baseline (speedup 1.0000x reference)
import jax
import jax.numpy as jnp
from jax.experimental import pallas as pl


def kernel(x, edge_index, Wl1, bl1, Wr1, Wl2, bl2, Wr2):
    raise NotImplementedError("write your pallas kernel here")


# speedup vs baseline: 10.2027x; 10.2027x over previous
import functools

import jax
import jax.numpy as jnp
from jax import lax
from jax.experimental import pallas as pl
from jax.experimental.pallas import tpu as pltpu
from jax.experimental.pallas import tpu_sc as plsc

N = 10000
E = 320000
D = 128
H1 = 16

NC = 2
NS = 16
NW = NC * NS
EPT = E // NW
W = 80
NWIN = EPT // W
ROWS_PER_TILE = N // NS


def _proj1_body(x_ref, wl_ref, wr_ref, y1_ref, z1_ref):
    x = x_ref[...]
    y1_ref[...] = lax.dot_general(x, wl_ref[...], (((1,), (1,)), ((), ())),
                                  preferred_element_type=jnp.float32)
    z1_ref[...] = lax.dot_general(x, wr_ref[...], (((1,), (1,)), ((), ())),
                                  preferred_element_type=jnp.float32)


def _proj1(x, Wl1, Wr1):
    return pl.pallas_call(
        _proj1_body,
        out_shape=[jax.ShapeDtypeStruct((N, H1), jnp.float32),
                   jax.ShapeDtypeStruct((N, H1), jnp.float32)],
    )(x, Wl1, Wr1)


def _sc_pass1_body(y1_hbm, src_hbm, dst_hbm, z2d_hbm, z1d_hbm,
                   agg_out, deg_out,
                   sbuf, dbuf, rbuf, ones_v, table_sh, accum_sh, deg_sh):
    c = lax.axis_index("c")
    s = lax.axis_index("s")
    wid = c * NS + s

    @pl.when(s == 0)
    def _():
        pltpu.sync_copy(y1_hbm, table_sh)
        pltpu.sync_copy(z2d_hbm, accum_sh)
        pltpu.sync_copy(z1d_hbm, deg_sh)

    for i in range(W // 16):
        ones_v[pl.ds(i * 16, 16)] = jnp.ones((16,), jnp.float32)

    plsc.subcore_barrier()

    def win(j, carry):
        base = wid * EPT + j * W
        pltpu.sync_copy(src_hbm.at[pl.ds(base, W)], sbuf)
        pltpu.sync_copy(dst_hbm.at[pl.ds(base, W)], dbuf)
        pltpu.sync_copy(table_sh.at[sbuf], rbuf)
        pltpu.sync_copy(ones_v, deg_sh.at[dbuf], add=True)
        pltpu.sync_copy(rbuf, accum_sh.at[dbuf], add=True)
        return carry

    lax.fori_loop(0, NWIN, win, 0)

    plsc.subcore_barrier()

    @pl.when(s == 0)
    def _():
        pltpu.sync_copy(accum_sh, agg_out.at[c])
        pltpu.sync_copy(deg_sh, deg_out.at[c])


def _sc_pass1(y1, src, dst, z2d, z1d):
    mesh = plsc.VectorSubcoreMesh(core_axis_name="c", subcore_axis_name="s")
    return pl.kernel(
        _sc_pass1_body,
        out_type=[jax.ShapeDtypeStruct((NC, N, H1), jnp.float32),
                  jax.ShapeDtypeStruct((NC, N), jnp.float32)],
        mesh=mesh,
        compiler_params=pltpu.CompilerParams(use_tc_tiling_on_sc=False),
        scratch_types=[
            pltpu.VMEM((W,), jnp.int32),
            pltpu.VMEM((W,), jnp.int32),
            pltpu.VMEM((W, H1), jnp.float32),
            pltpu.VMEM((W,), jnp.float32),
            pltpu.VMEM_SHARED((N, H1), jnp.float32),
            pltpu.VMEM_SHARED((N, H1), jnp.float32),
            pltpu.VMEM_SHARED((N,), jnp.float32),
        ],
    )(y1, src, dst, z2d, z1d)


def _mid_body(aggp_ref, degp_ref, z1_ref, bl1_ref, wl2_ref, wr2_ref, bl2_ref,
              y2_ref, z2_ref, inv_ref):
    deg = degp_ref[0] + degp_ref[1]
    inv = 1.0 / jnp.maximum(deg, 1.0)
    agg = aggp_ref[0] + aggp_ref[1]
    h = jnp.maximum(agg * inv[:, None] + bl1_ref[...] + z1_ref[...], 0.0)
    y2_ref[...] = jnp.sum(h * wl2_ref[...], axis=1)
    z2_ref[...] = jnp.sum(h * wr2_ref[...], axis=1) + bl2_ref[0]
    inv_ref[...] = inv


def _mid(aggp, degp, z1, bl1, Wl2, Wr2, bl2):
    return pl.pallas_call(
        _mid_body,
        out_shape=[jax.ShapeDtypeStruct((N,), jnp.float32),
                   jax.ShapeDtypeStruct((N,), jnp.float32),
                   jax.ShapeDtypeStruct((N,), jnp.float32)],
    )(aggp, degp, z1, bl1, Wl2, Wr2, bl2)


def _sc_pass2_body(y2_hbm, src_hbm, dst_hbm, z1d_hbm,
                   agg_out,
                   sbuf, dbuf, vbuf, table_sh, accum_sh):
    c = lax.axis_index("c")
    s = lax.axis_index("s")
    wid = c * NS + s

    @pl.when(s == 0)
    def _():
        pltpu.sync_copy(y2_hbm, table_sh)
        pltpu.sync_copy(z1d_hbm, accum_sh)

    plsc.subcore_barrier()

    def win(j, carry):
        base = wid * EPT + j * W
        pltpu.sync_copy(src_hbm.at[pl.ds(base, W)], sbuf)
        pltpu.sync_copy(dst_hbm.at[pl.ds(base, W)], dbuf)
        pltpu.sync_copy(table_sh.at[sbuf], vbuf)
        pltpu.sync_copy(vbuf, accum_sh.at[dbuf], add=True)
        return carry

    lax.fori_loop(0, NWIN, win, 0)

    plsc.subcore_barrier()

    @pl.when(s == 0)
    def _():
        pltpu.sync_copy(accum_sh, agg_out.at[c])


def _sc_pass2(y2, src, dst, z1d):
    mesh = plsc.VectorSubcoreMesh(core_axis_name="c", subcore_axis_name="s")
    return pl.kernel(
        _sc_pass2_body,
        out_type=jax.ShapeDtypeStruct((NC, N), jnp.float32),
        mesh=mesh,
        compiler_params=pltpu.CompilerParams(use_tc_tiling_on_sc=False),
        scratch_types=[
            pltpu.VMEM((W,), jnp.int32),
            pltpu.VMEM((W,), jnp.int32),
            pltpu.VMEM((W,), jnp.float32),
            pltpu.VMEM_SHARED((N,), jnp.float32),
            pltpu.VMEM_SHARED((N,), jnp.float32),
        ],
    )(y2, src, dst, z1d)


def _final_body(agg2p_ref, inv_ref, z2_ref, o_ref):
    o_ref[...] = (agg2p_ref[0] + agg2p_ref[1]) * inv_ref[...] + z2_ref[...]


def _final(agg2p, inv, z2):
    return pl.pallas_call(
        _final_body,
        out_shape=jax.ShapeDtypeStruct((N,), jnp.float32),
    )(agg2p, inv, z2)


def kernel(x, edge_index, Wl1, bl1, Wr1, Wl2, bl2, Wr2):
    src = edge_index[0]
    dst = edge_index[1]
    z2d = jnp.zeros((N, H1), jnp.float32)
    z1d = jnp.zeros((N,), jnp.float32)

    y1, z1 = _proj1(x, Wl1, Wr1)
    aggp, degp = _sc_pass1(y1, src, dst, z2d, z1d)
    y2, z2, inv = _mid(aggp, degp, z1, bl1, Wl2, Wr2, bl2)
    agg2p = _sc_pass2(y2, src, dst, z1d)
    out = _final(agg2p, inv, z2)
    return out.reshape(N, 1)


# speedup vs baseline: 26.1375x; 2.5618x over previous
import functools

import jax
import jax.numpy as jnp
from jax import lax
from jax.experimental import pallas as pl
from jax.experimental.pallas import tpu as pltpu
from jax.experimental.pallas import tpu_sc as plsc

N = 10000
E = 320000
D = 128
H1 = 16

NC = 2
NS = 16
NW = NC * NS
EPT = E // NW
W = 80
NWIN = EPT // W
K = 5
G = NWIN // K
NROW = E // W


def _proj1_body(x_ref, wl_ref, wr_ref, y1_ref, z1_ref):
    x = x_ref[...]
    y1_ref[...] = lax.dot_general(x, wl_ref[...], (((1,), (1,)), ((), ())),
                                  preferred_element_type=jnp.float32)
    z1_ref[...] = lax.dot_general(x, wr_ref[...], (((1,), (1,)), ((), ())),
                                  preferred_element_type=jnp.float32)


def _proj1(x, Wl1, Wr1):
    return pl.pallas_call(
        _proj1_body,
        out_shape=[jax.ShapeDtypeStruct((N, H1), jnp.float32),
                   jax.ShapeDtypeStruct((N, H1), jnp.float32)],
    )(x, Wl1, Wr1)


def _sc_pass1_body(y1_hbm, src2_hbm, dst2_hbm, z2d_hbm, z1d_hbm,
                   agg_out, deg_out,
                   sbuf, dbuf, rbuf, ones_v, table_sh, accum_sh, deg_sh,
                   sem_i, sem_g, sem_o, sem_s):
    c = lax.axis_index("c")
    s = lax.axis_index("s")
    wid = c * NS + s

    @pl.when(s == 0)
    def _():
        pltpu.sync_copy(y1_hbm, table_sh)
        pltpu.sync_copy(z2d_hbm, accum_sh)
        pltpu.sync_copy(z1d_hbm, deg_sh)

    for i in range(W // 16):
        ones_v[pl.ds(i * 16, 16)] = jnp.ones((16,), jnp.float32)

    plsc.subcore_barrier()

    def grp(g, carry):
        row0 = wid * NWIN + g * K
        di1 = pltpu.async_copy(src2_hbm.at[pl.ds(row0, K)], sbuf, sem_i)
        di2 = pltpu.async_copy(dst2_hbm.at[pl.ds(row0, K)], dbuf, sem_i)
        di1.wait()
        di2.wait()
        gds = [pltpu.async_copy(table_sh.at[sbuf.at[i]], rbuf.at[i], sem_g)
               for i in range(K)]
        ods = [pltpu.async_copy(ones_v, deg_sh.at[dbuf.at[i]], sem_o, add=True)
               for i in range(K)]
        for d in gds:
            d.wait()
        sds = [pltpu.async_copy(rbuf.at[i], accum_sh.at[dbuf.at[i]], sem_s,
                                add=True)
               for i in range(K)]
        for d in ods:
            d.wait()
        for d in sds:
            d.wait()
        return carry

    lax.fori_loop(0, G, grp, 0)

    plsc.subcore_barrier()

    @pl.when(s == 0)
    def _():
        pltpu.sync_copy(accum_sh, agg_out.at[c])
        pltpu.sync_copy(deg_sh, deg_out.at[c])


def _sc_pass1(y1, src2, dst2, z2d, z1d):
    mesh = plsc.VectorSubcoreMesh(core_axis_name="c", subcore_axis_name="s")
    return pl.kernel(
        _sc_pass1_body,
        out_type=[jax.ShapeDtypeStruct((NC, N, H1), jnp.float32),
                  jax.ShapeDtypeStruct((NC, N), jnp.float32)],
        mesh=mesh,
        compiler_params=pltpu.CompilerParams(use_tc_tiling_on_sc=False),
        scratch_types=[
            pltpu.VMEM((K, W), jnp.int32),
            pltpu.VMEM((K, W), jnp.int32),
            pltpu.VMEM((K, W, H1), jnp.float32),
            pltpu.VMEM((W,), jnp.float32),
            pltpu.VMEM_SHARED((N, H1), jnp.float32),
            pltpu.VMEM_SHARED((N, H1), jnp.float32),
            pltpu.VMEM_SHARED((N,), jnp.float32),
            pltpu.SemaphoreType.DMA,
            pltpu.SemaphoreType.DMA,
            pltpu.SemaphoreType.DMA,
            pltpu.SemaphoreType.DMA,
        ],
    )(y1, src2, dst2, z2d, z1d)


def _mid_body(aggp_ref, degp_ref, z1_ref, bl1_ref, wl2_ref, wr2_ref, bl2_ref,
              y2_ref, z2_ref, inv_ref):
    deg = degp_ref[0] + degp_ref[1]
    inv = 1.0 / jnp.maximum(deg, 1.0)
    agg = aggp_ref[0] + aggp_ref[1]
    h = jnp.maximum(agg * inv[:, None] + bl1_ref[...] + z1_ref[...], 0.0)
    y2_ref[...] = jnp.sum(h * wl2_ref[...], axis=1)
    z2_ref[...] = jnp.sum(h * wr2_ref[...], axis=1) + bl2_ref[0]
    inv_ref[...] = inv


def _mid(aggp, degp, z1, bl1, Wl2, Wr2, bl2):
    return pl.pallas_call(
        _mid_body,
        out_shape=[jax.ShapeDtypeStruct((N,), jnp.float32),
                   jax.ShapeDtypeStruct((N,), jnp.float32),
                   jax.ShapeDtypeStruct((N,), jnp.float32)],
    )(aggp, degp, z1, bl1, Wl2, Wr2, bl2)


def _sc_pass2_body(y2_hbm, src2_hbm, dst2_hbm, z1d_hbm,
                   agg_out,
                   sbuf, dbuf, vbuf, table_sh, accum_sh,
                   sem_i, sem_g, sem_s):
    c = lax.axis_index("c")
    s = lax.axis_index("s")
    wid = c * NS + s

    @pl.when(s == 0)
    def _():
        pltpu.sync_copy(y2_hbm, table_sh)
        pltpu.sync_copy(z1d_hbm, accum_sh)

    plsc.subcore_barrier()

    def grp(g, carry):
        row0 = wid * NWIN + g * K
        di1 = pltpu.async_copy(src2_hbm.at[pl.ds(row0, K)], sbuf, sem_i)
        di2 = pltpu.async_copy(dst2_hbm.at[pl.ds(row0, K)], dbuf, sem_i)
        di1.wait()
        di2.wait()
        gds = [pltpu.async_copy(table_sh.at[sbuf.at[i]], vbuf.at[i], sem_g)
               for i in range(K)]
        for d in gds:
            d.wait()
        sds = [pltpu.async_copy(vbuf.at[i], accum_sh.at[dbuf.at[i]], sem_s,
                                add=True)
               for i in range(K)]
        for d in sds:
            d.wait()
        return carry

    lax.fori_loop(0, G, grp, 0)

    plsc.subcore_barrier()

    @pl.when(s == 0)
    def _():
        pltpu.sync_copy(accum_sh, agg_out.at[c])


def _sc_pass2(y2, src2, dst2, z1d):
    mesh = plsc.VectorSubcoreMesh(core_axis_name="c", subcore_axis_name="s")
    return pl.kernel(
        _sc_pass2_body,
        out_type=jax.ShapeDtypeStruct((NC, N), jnp.float32),
        mesh=mesh,
        compiler_params=pltpu.CompilerParams(use_tc_tiling_on_sc=False),
        scratch_types=[
            pltpu.VMEM((K, W), jnp.int32),
            pltpu.VMEM((K, W), jnp.int32),
            pltpu.VMEM((K, W), jnp.float32),
            pltpu.VMEM_SHARED((N,), jnp.float32),
            pltpu.VMEM_SHARED((N,), jnp.float32),
            pltpu.SemaphoreType.DMA,
            pltpu.SemaphoreType.DMA,
            pltpu.SemaphoreType.DMA,
        ],
    )(y2, src2, dst2, z1d)


def _final_body(agg2p_ref, inv_ref, z2_ref, o_ref):
    o_ref[...] = (agg2p_ref[0] + agg2p_ref[1]) * inv_ref[...] + z2_ref[...]


def _final(agg2p, inv, z2):
    return pl.pallas_call(
        _final_body,
        out_shape=jax.ShapeDtypeStruct((N,), jnp.float32),
    )(agg2p, inv, z2)


def kernel(x, edge_index, Wl1, bl1, Wr1, Wl2, bl2, Wr2):
    src2 = edge_index[0].reshape(NROW, W)
    dst2 = edge_index[1].reshape(NROW, W)
    z2d = jnp.zeros((N, H1), jnp.float32)
    z1d = jnp.zeros((N,), jnp.float32)

    y1, z1 = _proj1(x, Wl1, Wr1)
    aggp, degp = _sc_pass1(y1, src2, dst2, z2d, z1d)
    y2, z2, inv = _mid(aggp, degp, z1, bl1, Wl2, Wr2, bl2)
    agg2p = _sc_pass2(y2, src2, dst2, z1d)
    out = _final(agg2p, inv, z2)
    return out.reshape(N, 1)


# speedup vs baseline: 33.7780x; 1.2923x over previous
import functools

import jax
import jax.numpy as jnp
from jax import lax
from jax.experimental import pallas as pl
from jax.experimental.pallas import tpu as pltpu
from jax.experimental.pallas import tpu_sc as plsc

N = 10000
E = 320000
D = 128
H1 = 16

NC = 2
NS = 16
NW = NC * NS
W = 128
NWIN = 80
EP = NW * NWIN * W
PAD = EP - E
NROWP = EP // W
K = 8
GB = NWIN // K
NP = N + 8
R8 = N // 8


def _proj1_body(x_ref, wl_ref, wr_ref, y1_ref, z1_ref):
    x = x_ref[...]
    y1_ref[...] = lax.dot_general(x, wl_ref[...], (((1,), (1,)), ((), ())),
                                  preferred_element_type=jnp.float32)
    z1_ref[...] = lax.dot_general(x, wr_ref[...], (((1,), (1,)), ((), ())),
                                  preferred_element_type=jnp.float32)


def _proj1(x, Wl1, Wr1):
    return pl.pallas_call(
        _proj1_body,
        out_shape=[jax.ShapeDtypeStruct((N, H1), jnp.float32),
                   jax.ShapeDtypeStruct((N, H1), jnp.float32)],
    )(x, Wl1, Wr1)


def _sc_pass1_body(y1_hbm, src2_hbm, dst2_hbm, z2d_hbm, z1d_hbm,
                   agg_out, deg_out,
                   sidx, didx, rbuf, ones_v, table_sh, accum_sh, deg_sh,
                   sem_i, sem_g, sem_o, sem_s):
    c = lax.axis_index("c")
    s = lax.axis_index("s")
    wid = c * NS + s

    @pl.when(s == 0)
    def _():
        pltpu.sync_copy(y1_hbm, table_sh)
        pltpu.sync_copy(z2d_hbm, accum_sh)
        pltpu.sync_copy(z1d_hbm, deg_sh)

    di1 = pltpu.async_copy(src2_hbm.at[pl.ds(wid * NWIN, NWIN)], sidx, sem_i)
    di2 = pltpu.async_copy(dst2_hbm.at[pl.ds(wid * NWIN, NWIN)], didx, sem_i)

    for i in range(W // 16):
        ones_v[pl.ds(i * 16, 16)] = jnp.ones((16,), jnp.float32)

    di1.wait()
    di2.wait()
    plsc.subcore_barrier()

    def batch(g, carry):
        base = g * K
        gds = [pltpu.async_copy(table_sh.at[sidx.at[base + i]], rbuf.at[i],
                                sem_g)
               for i in range(K)]
        ods = [pltpu.async_copy(ones_v, deg_sh.at[didx.at[base + i]], sem_o,
                                add=True)
               for i in range(K)]
        sds = []
        for i in range(K):
            gds[i].wait()
            sds.append(pltpu.async_copy(rbuf.at[i],
                                        accum_sh.at[didx.at[base + i]],
                                        sem_s, add=True))
        for d in ods:
            d.wait()
        for d in sds:
            d.wait()
        return carry

    lax.fori_loop(0, GB, batch, 0)

    plsc.subcore_barrier()

    @pl.when(s == 0)
    def _():
        pltpu.sync_copy(accum_sh.at[pl.ds(0, N)], agg_out.at[c])
        pltpu.sync_copy(deg_sh.at[pl.ds(0, N)], deg_out.at[c])


def _sc_pass1(y1, src2, dst2, z2d, z1d):
    mesh = plsc.VectorSubcoreMesh(core_axis_name="c", subcore_axis_name="s")
    return pl.kernel(
        _sc_pass1_body,
        out_type=[jax.ShapeDtypeStruct((NC, N, H1), jnp.float32),
                  jax.ShapeDtypeStruct((NC, N), jnp.float32)],
        mesh=mesh,
        compiler_params=pltpu.CompilerParams(use_tc_tiling_on_sc=False),
        scratch_types=[
            pltpu.VMEM((NWIN, W), jnp.int32),
            pltpu.VMEM((NWIN, W), jnp.int32),
            pltpu.VMEM((K, W, H1), jnp.float32),
            pltpu.VMEM((W,), jnp.float32),
            pltpu.VMEM_SHARED((N, H1), jnp.float32),
            pltpu.VMEM_SHARED((NP, H1), jnp.float32),
            pltpu.VMEM_SHARED((NP,), jnp.float32),
            pltpu.SemaphoreType.DMA,
            pltpu.SemaphoreType.DMA,
            pltpu.SemaphoreType.DMA,
            pltpu.SemaphoreType.DMA,
        ],
    )(y1, src2, dst2, z2d, z1d)


def _mid_body(aggp_ref, degp_ref, z1_ref, bl1_ref, exp_ref, bwl2_ref,
              bwr2_ref, bl2_ref, y2_ref, z2_ref, inv_ref):
    deg = degp_ref[0] + degp_ref[1]
    inv = 1.0 / jnp.maximum(deg, 1.0)
    invw = lax.dot_general(inv, exp_ref[...], (((1,), (0,)), ((), ())),
                           preferred_element_type=jnp.float32)
    agg = aggp_ref[0] + aggp_ref[1]
    h = jnp.maximum(agg * invw + bl1_ref[...] + z1_ref[...], 0.0)
    y2_ref[...] = lax.dot_general(h, bwl2_ref[...], (((1,), (0,)), ((), ())),
                                  preferred_element_type=jnp.float32)
    z2_ref[...] = lax.dot_general(h, bwr2_ref[...], (((1,), (0,)), ((), ())),
                                  preferred_element_type=jnp.float32) + bl2_ref[0]
    inv_ref[...] = inv


def _mid(aggp, degp, z1, bl1t, expm, Bwl2, Bwr2, bl2):
    return pl.pallas_call(
        _mid_body,
        out_shape=[jax.ShapeDtypeStruct((R8, 8), jnp.float32),
                   jax.ShapeDtypeStruct((R8, 8), jnp.float32),
                   jax.ShapeDtypeStruct((R8, 8), jnp.float32)],
    )(aggp, degp, z1, bl1t, expm, Bwl2, Bwr2, bl2)


def _sc_pass2_body(y2_hbm, src2_hbm, dst2_hbm, z1d_hbm,
                   agg_out,
                   sidx, didx, vbuf, table_sh, accum_sh,
                   sem_i, sem_g, sem_s):
    c = lax.axis_index("c")
    s = lax.axis_index("s")
    wid = c * NS + s

    @pl.when(s == 0)
    def _():
        pltpu.sync_copy(y2_hbm, table_sh)
        pltpu.sync_copy(z1d_hbm, accum_sh)

    di1 = pltpu.async_copy(src2_hbm.at[pl.ds(wid * NWIN, NWIN)], sidx, sem_i)
    di2 = pltpu.async_copy(dst2_hbm.at[pl.ds(wid * NWIN, NWIN)], didx, sem_i)
    di1.wait()
    di2.wait()
    plsc.subcore_barrier()

    def batch(g, carry):
        base = g * K
        gds = [pltpu.async_copy(table_sh.at[sidx.at[base + i]], vbuf.at[i],
                                sem_g)
               for i in range(K)]
        sds = []
        for i in range(K):
            gds[i].wait()
            sds.append(pltpu.async_copy(vbuf.at[i],
                                        accum_sh.at[didx.at[base + i]],
                                        sem_s, add=True))
        for d in sds:
            d.wait()
        return carry

    lax.fori_loop(0, GB, batch, 0)

    plsc.subcore_barrier()

    @pl.when(s == 0)
    def _():
        pltpu.sync_copy(accum_sh.at[pl.ds(0, N)], agg_out.at[c])


def _sc_pass2(y2, src2, dst2, z1d):
    mesh = plsc.VectorSubcoreMesh(core_axis_name="c", subcore_axis_name="s")
    return pl.kernel(
        _sc_pass2_body,
        out_type=jax.ShapeDtypeStruct((NC, N), jnp.float32),
        mesh=mesh,
        compiler_params=pltpu.CompilerParams(use_tc_tiling_on_sc=False),
        scratch_types=[
            pltpu.VMEM((NWIN, W), jnp.int32),
            pltpu.VMEM((NWIN, W), jnp.int32),
            pltpu.VMEM((K, W), jnp.float32),
            pltpu.VMEM_SHARED((N,), jnp.float32),
            pltpu.VMEM_SHARED((NP,), jnp.float32),
            pltpu.SemaphoreType.DMA,
            pltpu.SemaphoreType.DMA,
            pltpu.SemaphoreType.DMA,
        ],
    )(y2, src2, dst2, z1d)


def _final_body(agg2p_ref, inv_ref, z2_ref, o_ref):
    o_ref[...] = (agg2p_ref[0] + agg2p_ref[1]) * inv_ref[...] + z2_ref[...]


def _final(agg2p, inv, z2):
    return pl.pallas_call(
        _final_body,
        out_shape=jax.ShapeDtypeStruct((R8, 8), jnp.float32),
    )(agg2p, inv, z2)


def kernel(x, edge_index, Wl1, bl1, Wr1, Wl2, bl2, Wr2):
    src = edge_index[0]
    dst = edge_index[1]
    src2 = jnp.concatenate(
        [src, jnp.zeros((PAD,), jnp.int32)]).reshape(NROWP, W)
    dst2 = jnp.concatenate(
        [dst, N + (jnp.arange(PAD, dtype=jnp.int32) % 8)]).reshape(NROWP, W)
    z2d = jnp.zeros((NP, H1), jnp.float32)
    z1d = jnp.zeros((NP,), jnp.float32)

    lanes = jnp.arange(W, dtype=jnp.int32)
    blk = lanes[:, None] // H1 == jnp.arange(8, dtype=jnp.int32)[None, :]
    expm = blk.astype(jnp.float32).T
    Bwl2 = jnp.where(blk, jnp.tile(Wl2[0], 8)[:, None], 0.0)
    Bwr2 = jnp.where(blk, jnp.tile(Wr2[0], 8)[:, None], 0.0)
    bl1t = jnp.tile(bl1, 8)

    y1, z1 = _proj1(x, Wl1, Wr1)
    aggp, degp = _sc_pass1(y1, src2, dst2, z2d, z1d)
    y2, z2, inv = _mid(aggp.reshape(NC, R8, D), degp.reshape(NC, R8, 8),
                       z1.reshape(R8, D), bl1t, expm, Bwl2, Bwr2, bl2)
    agg2p = _sc_pass2(y2.reshape(N), src2, dst2, z1d)
    out = _final(agg2p.reshape(NC, R8, 8), inv, z2)
    return out.reshape(N, 1)


# speedup vs baseline: 34.6102x; 1.0246x over previous
import functools

import jax
import jax.numpy as jnp
from jax import lax
from jax.experimental import pallas as pl
from jax.experimental.pallas import tpu as pltpu
from jax.experimental.pallas import tpu_sc as plsc

N = 10000
E = 320000
D = 128
H1 = 16

NC = 2
NS = 16
NW = NC * NS
W = 128
NWIN = 80
EP = NW * NWIN * W
PAD = EP - E
NROWP = EP // W
K = 8
GB = NWIN // K
NP = N + 8
R8 = N // 8
CH = 624
CHL = N - (NS - 1) * CH
CHLP = NP - (NS - 1) * CH


def _proj1_body(x_ref, wl_ref, wr_ref, y1_ref, z1_ref):
    x = x_ref[...]
    y1_ref[...] = lax.dot_general(x, wl_ref[...], (((1,), (1,)), ((), ())),
                                  preferred_element_type=jnp.float32)
    z1_ref[...] = lax.dot_general(x, wr_ref[...], (((1,), (1,)), ((), ())),
                                  preferred_element_type=jnp.float32)


def _proj1(x, Wl1, Wr1):
    return pl.pallas_call(
        _proj1_body,
        out_shape=[jax.ShapeDtypeStruct((N, H1), jnp.float32),
                   jax.ShapeDtypeStruct((N, H1), jnp.float32)],
    )(x, Wl1, Wr1)


def _sc_pass1_body(y1_hbm, src2_hbm, dst2_hbm, z2d_hbm, z1d_hbm,
                   agg_out, deg_out,
                   sidx, didx, rbuf, ones_v, table_sh, accum_sh, deg_sh,
                   sem_i, sem_g, sem_o, sem_s):
    c = lax.axis_index("c")
    s = lax.axis_index("s")
    wid = c * NS + s

    di1 = pltpu.async_copy(src2_hbm.at[pl.ds(wid * NWIN, NWIN)], sidx, sem_i)
    di2 = pltpu.async_copy(dst2_hbm.at[pl.ds(wid * NWIN, NWIN)], didx, sem_i)

    r0 = s * CH

    @pl.when(s < NS - 1)
    def _():
        pltpu.sync_copy(y1_hbm.at[pl.ds(r0, CH)], table_sh.at[pl.ds(r0, CH)])
        pltpu.sync_copy(z2d_hbm.at[pl.ds(r0, CH)], accum_sh.at[pl.ds(r0, CH)])
        pltpu.sync_copy(z1d_hbm.at[pl.ds(r0, CH)], deg_sh.at[pl.ds(r0, CH)])

    @pl.when(s == NS - 1)
    def _():
        pltpu.sync_copy(y1_hbm.at[pl.ds(r0, CHL)], table_sh.at[pl.ds(r0, CHL)])
        pltpu.sync_copy(z2d_hbm.at[pl.ds(r0, CHLP)],
                        accum_sh.at[pl.ds(r0, CHLP)])
        pltpu.sync_copy(z1d_hbm.at[pl.ds(r0, CHLP)],
                        deg_sh.at[pl.ds(r0, CHLP)])

    for i in range(W // 16):
        ones_v[pl.ds(i * 16, 16)] = jnp.ones((16,), jnp.float32)

    di1.wait()
    di2.wait()
    plsc.subcore_barrier()

    def batch(g, carry):
        base = g * K
        gds = [pltpu.async_copy(table_sh.at[sidx.at[base + i]], rbuf.at[i],
                                sem_g)
               for i in range(K)]
        ods = [pltpu.async_copy(ones_v, deg_sh.at[didx.at[base + i]], sem_o,
                                add=True)
               for i in range(K)]
        sds = []
        for i in range(K):
            gds[i].wait()
            sds.append(pltpu.async_copy(rbuf.at[i],
                                        accum_sh.at[didx.at[base + i]],
                                        sem_s, add=True))
        for d in ods:
            d.wait()
        for d in sds:
            d.wait()
        return carry

    lax.fori_loop(0, GB, batch, 0)

    plsc.subcore_barrier()

    @pl.when(s < NS - 1)
    def _():
        pltpu.sync_copy(accum_sh.at[pl.ds(r0, CH)],
                        agg_out.at[c, pl.ds(r0, CH)])
        pltpu.sync_copy(deg_sh.at[pl.ds(r0, CH)],
                        deg_out.at[c, pl.ds(r0, CH)])

    @pl.when(s == NS - 1)
    def _():
        pltpu.sync_copy(accum_sh.at[pl.ds(r0, CHL)],
                        agg_out.at[c, pl.ds(r0, CHL)])
        pltpu.sync_copy(deg_sh.at[pl.ds(r0, CHL)],
                        deg_out.at[c, pl.ds(r0, CHL)])


def _sc_pass1(y1, src2, dst2, z2d, z1d):
    mesh = plsc.VectorSubcoreMesh(core_axis_name="c", subcore_axis_name="s")
    return pl.kernel(
        _sc_pass1_body,
        out_type=[jax.ShapeDtypeStruct((NC, N, H1), jnp.float32),
                  jax.ShapeDtypeStruct((NC, N), jnp.float32)],
        mesh=mesh,
        compiler_params=pltpu.CompilerParams(use_tc_tiling_on_sc=False),
        scratch_types=[
            pltpu.VMEM((NWIN, W), jnp.int32),
            pltpu.VMEM((NWIN, W), jnp.int32),
            pltpu.VMEM((K, W, H1), jnp.float32),
            pltpu.VMEM((W,), jnp.float32),
            pltpu.VMEM_SHARED((N, H1), jnp.float32),
            pltpu.VMEM_SHARED((NP, H1), jnp.float32),
            pltpu.VMEM_SHARED((NP,), jnp.float32),
            pltpu.SemaphoreType.DMA,
            pltpu.SemaphoreType.DMA,
            pltpu.SemaphoreType.DMA,
            pltpu.SemaphoreType.DMA,
        ],
    )(y1, src2, dst2, z2d, z1d)


def _mid_body(aggp_ref, degp_ref, z1_ref, bl1_ref, exp_ref, bwl2_ref,
              bwr2_ref, bl2_ref, y2_ref, z2_ref, inv_ref):
    deg = degp_ref[0] + degp_ref[1]
    inv = 1.0 / jnp.maximum(deg, 1.0)
    invw = lax.dot_general(inv, exp_ref[...], (((1,), (0,)), ((), ())),
                           preferred_element_type=jnp.float32)
    agg = aggp_ref[0] + aggp_ref[1]
    h = jnp.maximum(agg * invw + bl1_ref[...] + z1_ref[...], 0.0)
    y2_ref[...] = lax.dot_general(h, bwl2_ref[...], (((1,), (0,)), ((), ())),
                                  preferred_element_type=jnp.float32)
    z2_ref[...] = lax.dot_general(h, bwr2_ref[...], (((1,), (0,)), ((), ())),
                                  preferred_element_type=jnp.float32) + bl2_ref[0]
    inv_ref[...] = inv


def _mid(aggp, degp, z1, bl1t, expm, Bwl2, Bwr2, bl2):
    return pl.pallas_call(
        _mid_body,
        out_shape=[jax.ShapeDtypeStruct((R8, 8), jnp.float32),
                   jax.ShapeDtypeStruct((R8, 8), jnp.float32),
                   jax.ShapeDtypeStruct((R8, 8), jnp.float32)],
    )(aggp, degp, z1, bl1t, expm, Bwl2, Bwr2, bl2)


def _sc_pass2_body(y2_hbm, src2_hbm, dst2_hbm, z1d_hbm,
                   agg_out,
                   sidx, didx, vbuf, table_sh, accum_sh,
                   sem_i, sem_g, sem_s):
    c = lax.axis_index("c")
    s = lax.axis_index("s")
    wid = c * NS + s

    di1 = pltpu.async_copy(src2_hbm.at[pl.ds(wid * NWIN, NWIN)], sidx, sem_i)
    di2 = pltpu.async_copy(dst2_hbm.at[pl.ds(wid * NWIN, NWIN)], didx, sem_i)

    r0 = s * CH

    @pl.when(s < NS - 1)
    def _():
        pltpu.sync_copy(y2_hbm.at[pl.ds(r0, CH)], table_sh.at[pl.ds(r0, CH)])
        pltpu.sync_copy(z1d_hbm.at[pl.ds(r0, CH)], accum_sh.at[pl.ds(r0, CH)])

    @pl.when(s == NS - 1)
    def _():
        pltpu.sync_copy(y2_hbm.at[pl.ds(r0, CHL)], table_sh.at[pl.ds(r0, CHL)])
        pltpu.sync_copy(z1d_hbm.at[pl.ds(r0, CHLP)],
                        accum_sh.at[pl.ds(r0, CHLP)])

    di1.wait()
    di2.wait()
    plsc.subcore_barrier()

    def batch(g, carry):
        base = g * K
        gds = [pltpu.async_copy(table_sh.at[sidx.at[base + i]], vbuf.at[i],
                                sem_g)
               for i in range(K)]
        sds = []
        for i in range(K):
            gds[i].wait()
            sds.append(pltpu.async_copy(vbuf.at[i],
                                        accum_sh.at[didx.at[base + i]],
                                        sem_s, add=True))
        for d in sds:
            d.wait()
        return carry

    lax.fori_loop(0, GB, batch, 0)

    plsc.subcore_barrier()

    @pl.when(s < NS - 1)
    def _():
        pltpu.sync_copy(accum_sh.at[pl.ds(r0, CH)],
                        agg_out.at[c, pl.ds(r0, CH)])

    @pl.when(s == NS - 1)
    def _():
        pltpu.sync_copy(accum_sh.at[pl.ds(r0, CHL)],
                        agg_out.at[c, pl.ds(r0, CHL)])


def _sc_pass2(y2, src2, dst2, z1d):
    mesh = plsc.VectorSubcoreMesh(core_axis_name="c", subcore_axis_name="s")
    return pl.kernel(
        _sc_pass2_body,
        out_type=jax.ShapeDtypeStruct((NC, N), jnp.float32),
        mesh=mesh,
        compiler_params=pltpu.CompilerParams(use_tc_tiling_on_sc=False),
        scratch_types=[
            pltpu.VMEM((NWIN, W), jnp.int32),
            pltpu.VMEM((NWIN, W), jnp.int32),
            pltpu.VMEM((K, W), jnp.float32),
            pltpu.VMEM_SHARED((N,), jnp.float32),
            pltpu.VMEM_SHARED((NP,), jnp.float32),
            pltpu.SemaphoreType.DMA,
            pltpu.SemaphoreType.DMA,
            pltpu.SemaphoreType.DMA,
        ],
    )(y2, src2, dst2, z1d)


def _final_body(agg2p_ref, inv_ref, z2_ref, o_ref):
    o_ref[...] = (agg2p_ref[0] + agg2p_ref[1]) * inv_ref[...] + z2_ref[...]


def _final(agg2p, inv, z2):
    return pl.pallas_call(
        _final_body,
        out_shape=jax.ShapeDtypeStruct((R8, 8), jnp.float32),
    )(agg2p, inv, z2)


def kernel(x, edge_index, Wl1, bl1, Wr1, Wl2, bl2, Wr2):
    src = edge_index[0]
    dst = edge_index[1]
    src2 = jnp.concatenate(
        [src, jnp.zeros((PAD,), jnp.int32)]).reshape(NROWP, W)
    dst2 = jnp.concatenate(
        [dst, N + (jnp.arange(PAD, dtype=jnp.int32) % 8)]).reshape(NROWP, W)
    z2d = jnp.zeros((NP, H1), jnp.float32)
    z1d = jnp.zeros((NP,), jnp.float32)

    lanes = jnp.arange(W, dtype=jnp.int32)
    blk = lanes[:, None] // H1 == jnp.arange(8, dtype=jnp.int32)[None, :]
    expm = blk.astype(jnp.float32).T
    Bwl2 = jnp.where(blk, jnp.tile(Wl2[0], 8)[:, None], 0.0)
    Bwr2 = jnp.where(blk, jnp.tile(Wr2[0], 8)[:, None], 0.0)
    bl1t = jnp.tile(bl1, 8)

    y1, z1 = _proj1(x, Wl1, Wr1)
    aggp, degp = _sc_pass1(y1, src2, dst2, z2d, z1d)
    y2, z2, inv = _mid(aggp.reshape(NC, R8, D), degp.reshape(NC, R8, 8),
                       z1.reshape(R8, D), bl1t, expm, Bwl2, Bwr2, bl2)
    agg2p = _sc_pass2(y2.reshape(N), src2, dst2, z1d)
    out = _final(agg2p.reshape(NC, R8, 8), inv, z2)
    return out.reshape(N, 1)
